# baseline (speedup 1.0000x reference)
import functools

import jax
import jax.numpy as jnp
from jax import lax
from jax.experimental import pallas as pl
from jax.experimental.pallas import tpu as pltpu
from jax.experimental.pallas import tpu_sc as plsc

B = 16384
D = 128
H1, H2, H3 = 1024, 512, 256
NW = 32
BPW = B // NW
BM = 2048

CH = 128
NCH = BPW // CH
NBUF = 4


@functools.partial(
    pl.kernel,
    mesh=plsc.VectorSubcoreMesh(core_axis_name="c", subcore_axis_name="s"),
    out_type=jax.ShapeDtypeStruct((B, 2 * D), jnp.float32),
    scratch_types=(
        [pltpu.VMEM((BPW,), jnp.int32) for _ in range(2)]
        + [pltpu.VMEM((CH, D), jnp.float32) for _ in range(NBUF)]
        + [pltpu.SemaphoreType.DMA for _ in range(2 * NBUF)]
    ),
)
def _gather_embeds(user_hbm, movie_hbm, utab_hbm, mtab_hbm,
                   x_hbm, idx_u, idx_m, *rest):
    bufs = rest[:NBUF]
    gsems = rest[NBUF:2 * NBUF]
    wsems = rest[2 * NBUF:]
    wid = lax.axis_index("s") * 2 + lax.axis_index("c")
    base = wid * BPW
    pltpu.sync_copy(user_hbm.at[pl.ds(base, BPW)], idx_u)
    pltpu.sync_copy(movie_hbm.at[pl.ds(base, BPW)], idx_m)

    jobs = [(t, ci) for ci in range(NCH) for t in range(2)]
    tabs = (utab_hbm, mtab_hbm)
    idxs = (idx_u, idx_m)
    nj = len(jobs)
    gh = [None] * nj
    wh = [None] * nj

    def start_gather(j):
        t, ci = jobs[j]
        b = j % NBUF
        return pltpu.async_copy(
            tabs[t].at[idxs[t].at[pl.ds(ci * CH, CH)]], bufs[b], gsems[b])

    def start_write(j):
        t, ci = jobs[j]
        b = j % NBUF
        return pltpu.async_copy(
            bufs[b],
            x_hbm.at[pl.ds(base + ci * CH, CH), pl.ds(t * D, D)],
            wsems[b])

    for j in range(nj):
        if j >= NBUF:
            wh[j - NBUF].wait()
        gh[j] = start_gather(j)
        if j >= 1:
            gh[j - 1].wait()
            wh[j - 1] = start_write(j - 1)
    gh[nj - 1].wait()
    wh[nj - 1] = start_write(nj - 1)
    for j in range(max(nj - NBUF, 0), nj):
        wh[j].wait()


def _mlp_body(x_in, w1, b1, w2, b2, w3, b3, w4t, b4, out):
    bf = jnp.bfloat16
    zero = jnp.zeros((), bf)
    x = jnp.dot(x_in[...].astype(bf), w1[...],
                preferred_element_type=jnp.float32)
    x = jnp.maximum(x.astype(bf) + b1[...], zero)
    x = jnp.dot(x, w2[...], preferred_element_type=jnp.float32)
    x = jnp.maximum(x.astype(bf) + b2[...], zero)
    x = jnp.dot(x, w3[...], preferred_element_type=jnp.float32)
    x = jnp.maximum(x.astype(bf) + b3[...], zero)
    out[...] = (jnp.sum(x.astype(jnp.float32) * w4t[...], axis=1,
                        keepdims=True) + b4[...])


def _mlp(x, W1, b1, W2, b2, W3, b3, W4, b4):
    n = x.shape[0]
    pcall = pl.pallas_call(
        _mlp_body,
        grid=(n // BM,),
        in_specs=[
            pl.BlockSpec((BM, 2 * D), lambda i: (i, 0)),
            pl.BlockSpec((2 * D, H1), lambda i: (0, 0)),
            pl.BlockSpec((1, H1), lambda i: (0, 0)),
            pl.BlockSpec((H1, H2), lambda i: (0, 0)),
            pl.BlockSpec((1, H2), lambda i: (0, 0)),
            pl.BlockSpec((H2, H3), lambda i: (0, 0)),
            pl.BlockSpec((1, H3), lambda i: (0, 0)),
            pl.BlockSpec((1, H3), lambda i: (0, 0)),
            pl.BlockSpec((1, 1), lambda i: (0, 0)),
        ],
        out_specs=pl.BlockSpec((BM, 1), lambda i: (i, 0)),
        out_shape=jax.ShapeDtypeStruct((n, 1), jnp.float32),
    )
    bf = jnp.bfloat16
    return pcall(x, W1.astype(bf), b1.reshape(1, H1).astype(bf),
                 W2.astype(bf), b2.reshape(1, H2).astype(bf),
                 W3.astype(bf), b3.reshape(1, H3).astype(bf),
                 W4.reshape(1, H3), b4.reshape(1, 1))


def kernel(user, movie, user_table, movie_table, W1, b1, W2, b2, W3, b3, W4, b4):
    u32 = user.astype(jnp.int32)
    m32 = movie.astype(jnp.int32)
    x = _gather_embeds(u32, m32, user_table, movie_table)
    return _mlp(x, W1, b1, W2, b2, W3, b3, W4, b4)


# speedup vs baseline: 1.0263x; 1.0263x over previous
import functools

import jax
import jax.numpy as jnp
from jax import lax
from jax.experimental import pallas as pl
from jax.experimental.pallas import tpu as pltpu
from jax.experimental.pallas import tpu_sc as plsc

B = 16384
D = 128
H1, H2, H3 = 1024, 512, 256
NW = 32
BM = 2048

CH = 128
NBUF = 4


def _make_gather(nrows):
    bpw = nrows // NW
    nch = bpw // CH

    @functools.partial(
        pl.kernel,
        mesh=plsc.VectorSubcoreMesh(core_axis_name="c", subcore_axis_name="s"),
        out_type=jax.ShapeDtypeStruct((nrows, 2 * D), jnp.float32),
        scratch_types=(
            [pltpu.VMEM((bpw,), jnp.int32) for _ in range(2)]
            + [pltpu.VMEM((CH, D), jnp.float32) for _ in range(NBUF)]
            + [pltpu.SemaphoreType.DMA for _ in range(2 * NBUF)]
        ),
    )
    def gather_embeds(user_hbm, movie_hbm, utab_hbm, mtab_hbm,
                      x_hbm, idx_u, idx_m, *rest):
        bufs = rest[:NBUF]
        gsems = rest[NBUF:2 * NBUF]
        wsems = rest[2 * NBUF:]
        wid = lax.axis_index("s") * 2 + lax.axis_index("c")
        base = wid * bpw
        pltpu.sync_copy(user_hbm.at[pl.ds(base, bpw)], idx_u)
        pltpu.sync_copy(movie_hbm.at[pl.ds(base, bpw)], idx_m)

        jobs = [(t, ci) for ci in range(nch) for t in range(2)]
        tabs = (utab_hbm, mtab_hbm)
        idxs = (idx_u, idx_m)
        nj = len(jobs)
        gh = [None] * nj
        wh = [None] * nj

        def start_gather(j):
            t, ci = jobs[j]
            b = j % NBUF
            return pltpu.async_copy(
                tabs[t].at[idxs[t].at[pl.ds(ci * CH, CH)]], bufs[b], gsems[b])

        def start_write(j):
            t, ci = jobs[j]
            b = j % NBUF
            return pltpu.async_copy(
                bufs[b],
                x_hbm.at[pl.ds(base + ci * CH, CH), pl.ds(t * D, D)],
                wsems[b])

        for j in range(nj):
            if j >= NBUF:
                wh[j - NBUF].wait()
            gh[j] = start_gather(j)
            if j >= 1:
                gh[j - 1].wait()
                wh[j - 1] = start_write(j - 1)
        gh[nj - 1].wait()
        wh[nj - 1] = start_write(nj - 1)
        for j in range(max(nj - NBUF, 0), nj):
            wh[j].wait()

    return gather_embeds


_gather_half = _make_gather(B // 2)


def _mlp_body(x_in, w1, b1, w2, b2, w3, b3, w4t, b4, out):
    bf = jnp.bfloat16
    zero = jnp.zeros((), bf)
    x = jnp.dot(x_in[...].astype(bf), w1[...],
                preferred_element_type=jnp.float32)
    x = jnp.maximum(x.astype(bf) + b1[...], zero)
    x = jnp.dot(x, w2[...], preferred_element_type=jnp.float32)
    x = jnp.maximum(x.astype(bf) + b2[...], zero)
    x = jnp.dot(x, w3[...], preferred_element_type=jnp.float32)
    x = jnp.maximum(x.astype(bf) + b3[...], zero)
    out[...] = (jnp.sum(x.astype(jnp.float32) * w4t[...], axis=1,
                        keepdims=True) + b4[...])


def _mlp(x, W1, b1, W2, b2, W3, b3, W4t, b4):
    n = x.shape[0]
    pcall = pl.pallas_call(
        _mlp_body,
        grid=(n // BM,),
        in_specs=[
            pl.BlockSpec((BM, 2 * D), lambda i: (i, 0)),
            pl.BlockSpec((2 * D, H1), lambda i: (0, 0)),
            pl.BlockSpec((1, H1), lambda i: (0, 0)),
            pl.BlockSpec((H1, H2), lambda i: (0, 0)),
            pl.BlockSpec((1, H2), lambda i: (0, 0)),
            pl.BlockSpec((H2, H3), lambda i: (0, 0)),
            pl.BlockSpec((1, H3), lambda i: (0, 0)),
            pl.BlockSpec((1, H3), lambda i: (0, 0)),
            pl.BlockSpec((1, 1), lambda i: (0, 0)),
        ],
        out_specs=pl.BlockSpec((BM, 1), lambda i: (i, 0)),
        out_shape=jax.ShapeDtypeStruct((n, 1), jnp.float32),
    )
    return pcall(x, W1, b1, W2, b2, W3, b3, W4t, b4)


def kernel(user, movie, user_table, movie_table, W1, b1, W2, b2, W3, b3, W4, b4):
    u32 = user.astype(jnp.int32)
    m32 = movie.astype(jnp.int32)
    bf = jnp.bfloat16
    args = (W1.astype(bf), b1.reshape(1, H1).astype(bf),
            W2.astype(bf), b2.reshape(1, H2).astype(bf),
            W3.astype(bf), b3.reshape(1, H3).astype(bf),
            W4.reshape(1, H3), b4.reshape(1, 1))
    h = B // 2
    x0 = _gather_half(u32[:h], m32[:h], user_table, movie_table)
    x1 = _gather_half(u32[h:], m32[h:], user_table, movie_table)
    y0 = _mlp(x0, *args)
    y1 = _mlp(x1, *args)
    return jnp.concatenate([y0, y1], axis=0)
